# Initial kernel scaffold; baseline (speedup 1.0000x reference)
import jax
import jax.numpy as jnp
from jax.experimental import pallas as pl


def kernel(Xv_in, v, e, W1, b1, W2, b2, gamma, beta):
    raise NotImplementedError("write your pallas kernel here")


# speedup vs baseline: 4.4060x; 4.4060x over previous
import functools

import jax
import jax.numpy as jnp
from jax import lax
from jax.experimental import pallas as pl
from jax.experimental.pallas import tpu as pltpu
from jax.experimental.pallas import tpu_sc as plsc

NW = 32
CH = 80
ZB = 40
VT = 10240
TRASH = 10200


def _sc_gather(table, idx):
    V, Dw = table.shape
    B = idx.shape[0]
    bpw = B // NW
    nch = bpw // CH
    mesh = plsc.VectorSubcoreMesh(core_axis_name="c", subcore_axis_name="s")

    @functools.partial(
        pl.kernel, mesh=mesh,
        out_type=jax.ShapeDtypeStruct((B, Dw), jnp.float32),
        scratch_types=[
            pltpu.VMEM((bpw,), jnp.int32),
            pltpu.VMEM((CH, Dw), jnp.float32),
            pltpu.SemaphoreType.DMA,
        ],
    )
    def k(table_hbm, idx_hbm, out_hbm, idx_v, rows_v, sem):
        wid = lax.axis_index("s") * 2 + lax.axis_index("c")
        base = wid * bpw
        pltpu.sync_copy(idx_hbm.at[pl.ds(base, bpw)], idx_v)

        def body(j, carry):
            pltpu.async_copy(
                table_hbm.at[idx_v.at[pl.ds(j * CH, CH)]], rows_v, sem
            ).wait()
            pltpu.sync_copy(rows_v, out_hbm.at[pl.ds(base + j * CH, CH)])
            return carry

        lax.fori_loop(0, nch, body, 0)

    return k(table, idx)


def _sc_scatter_add(rows, idx3):
    B, Dw = rows.shape
    bpw = B // NW
    nch = bpw // CH
    rpt = VT // 16
    nzb = rpt // ZB
    mesh = plsc.VectorSubcoreMesh(core_axis_name="c", subcore_axis_name="s")

    @functools.partial(
        pl.kernel, mesh=mesh,
        out_type=jax.ShapeDtypeStruct((2, VT, Dw), jnp.float32),
        scratch_types=[
            pltpu.VMEM((nch, CH), jnp.int32),
            pltpu.VMEM((CH, Dw), jnp.float32),
            pltpu.VMEM((ZB, Dw), jnp.float32),
            pltpu.VMEM_SHARED((VT, Dw), jnp.float32),
        ],
    )
    def k(rows_hbm, idx_hbm, out_hbm, idx_v, rbuf, zbuf, table):
        cid = lax.axis_index("c")
        sid = lax.axis_index("s")
        wid = sid * 2 + cid
        base = wid * bpw
        tbase = sid * rpt

        def zrow(i, carry):
            def zcol(j, c2):
                zbuf[i, pl.ds(j * 16, 16)] = jnp.zeros((16,), jnp.float32)
                return c2
            lax.fori_loop(0, Dw // 16, zcol, 0)
            return carry

        lax.fori_loop(0, ZB, zrow, 0)

        def ztab(i, carry):
            pltpu.sync_copy(zbuf, table.at[pl.ds(tbase + i * ZB, ZB)])
            return carry

        lax.fori_loop(0, nzb, ztab, 0)
        plsc.subcore_barrier()

        pltpu.sync_copy(idx_hbm.at[wid], idx_v)

        def body(j, carry):
            pltpu.sync_copy(rows_hbm.at[pl.ds(base + j * CH, CH)], rbuf)
            pltpu.sync_copy(rbuf, table.at[idx_v.at[j]], add=True)
            return carry

        lax.fori_loop(0, nch, body, 0)
        plsc.subcore_barrier()

        def dump(i, carry):
            pltpu.sync_copy(table.at[pl.ds(tbase + i * ZB, ZB)], zbuf)
            pltpu.sync_copy(zbuf, out_hbm.at[cid, pl.ds(tbase + i * ZB, ZB)])
            return carry

        lax.fori_loop(0, nzb, dump, 0)

    return k(rows, idx3)


def _tc_logits1(G, W1, b1):
    B = 2000
    n = G.shape[0] // B

    def body(g_ref, w_ref, b_ref, eg_ref, eb_ref):
        g = g_ref[...]
        a1 = jnp.dot(g, w_ref[...], preferred_element_type=jnp.float32)
        ex = jnp.exp(a1 + b_ref[0, 0])
        eg_ref[...] = ex * g
        eb_ref[...] = jnp.broadcast_to(ex, (B, 128))

    return pl.pallas_call(
        body,
        grid=(n,),
        in_specs=[
            pl.BlockSpec((B, 128), lambda i: (i, 0)),
            pl.BlockSpec((128, 1), lambda i: (0, 0)),
            pl.BlockSpec((1, 1), lambda i: (0, 0)),
        ],
        out_specs=[
            pl.BlockSpec((B, 128), lambda i: (i, 0)),
            pl.BlockSpec((B, 128), lambda i: (i, 0)),
        ],
        out_shape=[
            jax.ShapeDtypeStruct((G.shape[0], 128), jnp.float32),
            jax.ShapeDtypeStruct((G.shape[0], 128), jnp.float32),
        ],
    )(G, W1, b1)


def _tc_edge_norm(Sg, Sb, gamma, beta):
    M = Sg.shape[1]
    B = 400
    n = M // B

    def body(sg_ref, sb_ref, g_ref, be_ref, t2_ref):
        num = sg_ref[0] + sg_ref[1]
        s1 = sb_ref[0, :, 0:1] + sb_ref[1, :, 0:1]
        mu = jnp.mean(num, axis=1, keepdims=True)
        xc = num - mu
        var = jnp.mean(xc * xc, axis=1, keepdims=True)
        y = xc * lax.rsqrt(var + 1e-5) * g_ref[...] + be_ref[...]
        t2_ref[:, :128] = y
        t2_ref[:, 128:256] = jnp.broadcast_to(s1, (B, 128))

    return pl.pallas_call(
        body,
        grid=(n,),
        in_specs=[
            pl.BlockSpec((2, B, 128), lambda i: (0, i, 0)),
            pl.BlockSpec((2, B, 128), lambda i: (0, i, 0)),
            pl.BlockSpec((1, 128), lambda i: (0, 0)),
            pl.BlockSpec((1, 128), lambda i: (0, 0)),
        ],
        out_specs=pl.BlockSpec((B, 256), lambda i: (i, 0)),
        out_shape=jax.ShapeDtypeStruct((M, 256), jnp.float32),
    )(Sg, Sb, gamma, beta)


def _tc_logits2(EG, XA, W2, b2):
    B = 2000
    n = EG.shape[0] // B

    def body(eg_ref, xa_ref, w_ref, b_ref, r2_ref, rb_ref):
        xev = xa_ref[:, :128]
        s1g = xa_ref[:, 128:129] + 1e-16
        u = eg_ref[...]
        w = w_ref[...]
        a2 = (
            jnp.dot(xev, w[:128, :], preferred_element_type=jnp.float32)
            + jnp.dot(u, w[128:, :], preferred_element_type=jnp.float32) / s1g
            + b_ref[0, 0]
        )
        ex2 = jnp.exp(a2)
        r2_ref[...] = ex2 * xev
        rb_ref[...] = jnp.broadcast_to(ex2, (B, 128))

    return pl.pallas_call(
        body,
        grid=(n,),
        in_specs=[
            pl.BlockSpec((B, 128), lambda i: (i, 0)),
            pl.BlockSpec((B, 256), lambda i: (i, 0)),
            pl.BlockSpec((256, 1), lambda i: (0, 0)),
            pl.BlockSpec((1, 1), lambda i: (0, 0)),
        ],
        out_specs=[
            pl.BlockSpec((B, 128), lambda i: (i, 0)),
            pl.BlockSpec((B, 128), lambda i: (i, 0)),
        ],
        out_shape=[
            jax.ShapeDtypeStruct((EG.shape[0], 128), jnp.float32),
            jax.ShapeDtypeStruct((EG.shape[0], 128), jnp.float32),
        ],
    )(EG, XA, W2, b2)


def _tc_final(Sv, SvB, Xv_in):
    N = Xv_in.shape[0]
    B = 400
    n = N // B

    def body(s_ref, sb_ref, xv_ref, out_ref):
        num = s_ref[0] + s_ref[1]
        den = sb_ref[0, :, 0:1] + sb_ref[1, :, 0:1] + 1e-16
        out_ref[...] = num / den + xv_ref[...]

    return pl.pallas_call(
        body,
        grid=(n,),
        in_specs=[
            pl.BlockSpec((2, B, 128), lambda i: (0, i, 0)),
            pl.BlockSpec((2, B, 128), lambda i: (0, i, 0)),
            pl.BlockSpec((B, 128), lambda i: (i, 0)),
        ],
        out_specs=pl.BlockSpec((B, 128), lambda i: (i, 0)),
        out_shape=jax.ShapeDtypeStruct((N, 128), jnp.float32),
    )(Sv, SvB, Xv_in)


def kernel(Xv_in, v, e, W1, b1, W2, b2, gamma, beta):
    N, D = Xv_in.shape
    NNZ = v.shape[0]
    M = 20000
    MH = M // 2
    bpw = NNZ // NW
    nch = bpw // CH

    e0 = jnp.where(e < MH, e, TRASH).reshape(NW, nch, CH)
    e1 = jnp.where(e >= MH, e - MH, TRASH).reshape(NW, nch, CH)
    v3 = v.reshape(NW, nch, CH)

    G = _sc_gather(Xv_in, v)
    EG, EB = _tc_logits1(G, W1, jnp.reshape(b1, (1, 1)))

    Sg = jnp.concatenate(
        [_sc_scatter_add(EG, e0)[:, :MH], _sc_scatter_add(EG, e1)[:, :MH]],
        axis=1)
    Sb = jnp.concatenate(
        [_sc_scatter_add(EB, e0)[:, :MH], _sc_scatter_add(EB, e1)[:, :MH]],
        axis=1)

    T2 = _tc_edge_norm(Sg, Sb, jnp.reshape(gamma, (1, 128)),
                       jnp.reshape(beta, (1, 128)))
    XA = _sc_gather(T2, e)
    R2, R2B = _tc_logits2(EG, XA, jnp.reshape(W2, (256, 1)),
                          jnp.reshape(b2, (1, 1)))

    Sv = _sc_scatter_add(R2, v3)[:, :N]
    SvB = _sc_scatter_add(R2B, v3)[:, :N]
    return _tc_final(Sv, SvB, Xv_in)


# speedup vs baseline: 4.4406x; 1.0079x over previous
import functools

import jax
import jax.numpy as jnp
from jax import lax
from jax.experimental import pallas as pl
from jax.experimental.pallas import tpu as pltpu
from jax.experimental.pallas import tpu_sc as plsc

NW = 32
CH = 80
ZB = 40
VT = 10240
TRASH = 10200


def _sc_gather(table, idx):
    V, Dw = table.shape
    B = idx.shape[0]
    bpw = B // NW
    nch = bpw // CH
    mesh = plsc.VectorSubcoreMesh(core_axis_name="c", subcore_axis_name="s")

    @functools.partial(
        pl.kernel, mesh=mesh,
        out_type=jax.ShapeDtypeStruct((B, Dw), jnp.float32),
        scratch_types=[
            pltpu.VMEM((bpw,), jnp.int32),
            pltpu.VMEM((CH, Dw), jnp.float32),
            pltpu.SemaphoreType.DMA,
        ],
    )
    def k(table_hbm, idx_hbm, out_hbm, idx_v, rows_v, sem):
        wid = lax.axis_index("s") * 2 + lax.axis_index("c")
        base = wid * bpw
        pltpu.sync_copy(idx_hbm.at[pl.ds(base, bpw)], idx_v)

        def body(j, carry):
            pltpu.async_copy(
                table_hbm.at[idx_v.at[pl.ds(j * CH, CH)]], rows_v, sem
            ).wait()
            pltpu.sync_copy(rows_v, out_hbm.at[pl.ds(base + j * CH, CH)])
            return carry

        lax.fori_loop(0, nch, body, 0)

    return k(table, idx)


def _chunk_ranges(idx3):
    act = jnp.any(idx3 < TRASH, axis=2)
    cnt = jnp.sum(act.astype(jnp.int32), axis=1)
    lo = jnp.where(cnt > 0, jnp.argmax(act, axis=1).astype(jnp.int32), 0)
    return jnp.pad(jnp.stack([lo, lo + cnt], axis=1), ((0, 0), (0, 14)))


def _sc_scatter_add(rows, idx3, rng):
    B, Dw = rows.shape
    bpw = B // NW
    nch = bpw // CH
    rpt = VT // 16
    nzb = rpt // ZB
    mesh = plsc.VectorSubcoreMesh(core_axis_name="c", subcore_axis_name="s")

    @functools.partial(
        pl.kernel, mesh=mesh,
        out_type=jax.ShapeDtypeStruct((2, VT, Dw), jnp.float32),
        scratch_types=[
            pltpu.VMEM((nch, CH), jnp.int32),
            pltpu.VMEM((16,), jnp.int32),
            pltpu.VMEM((CH, Dw), jnp.float32),
            pltpu.VMEM((ZB, Dw), jnp.float32),
            pltpu.VMEM_SHARED((VT, Dw), jnp.float32),
        ],
    )
    def k(rows_hbm, idx_hbm, rng_hbm, out_hbm, idx_v, rng_v, rbuf, zbuf,
          table):
        cid = lax.axis_index("c")
        sid = lax.axis_index("s")
        wid = sid * 2 + cid
        base = wid * bpw
        tbase = sid * rpt

        def zrow(i, carry):
            def zcol(j, c2):
                zbuf[i, pl.ds(j * 16, 16)] = jnp.zeros((16,), jnp.float32)
                return c2
            lax.fori_loop(0, Dw // 16, zcol, 0)
            return carry

        lax.fori_loop(0, ZB, zrow, 0)

        def ztab(i, carry):
            pltpu.sync_copy(zbuf, table.at[pl.ds(tbase + i * ZB, ZB)])
            return carry

        lax.fori_loop(0, nzb, ztab, 0)
        plsc.subcore_barrier()

        pltpu.sync_copy(idx_hbm.at[wid], idx_v)
        pltpu.sync_copy(rng_hbm.at[pl.ds(wid * 16, 16)], rng_v)
        rng = rng_v[...]

        def body(j, carry):
            pltpu.sync_copy(rows_hbm.at[pl.ds(base + j * CH, CH)], rbuf)
            pltpu.sync_copy(rbuf, table.at[idx_v.at[j]], add=True)
            return carry

        lax.fori_loop(rng[0], rng[1], body, 0)
        plsc.subcore_barrier()

        def dump(i, carry):
            pltpu.sync_copy(table.at[pl.ds(tbase + i * ZB, ZB)], zbuf)
            pltpu.sync_copy(zbuf, out_hbm.at[cid, pl.ds(tbase + i * ZB, ZB)])
            return carry

        lax.fori_loop(0, nzb, dump, 0)

    return k(rows, idx3, rng.reshape(-1))


def _tc_logits1(G, W1, b1):
    B = 2000
    n = G.shape[0] // B

    def body(g_ref, w_ref, b_ref, eg_ref, eb_ref):
        g = g_ref[...]
        a1 = jnp.dot(g, w_ref[...], preferred_element_type=jnp.float32)
        ex = jnp.exp(a1 + b_ref[0, 0])
        eg_ref[...] = ex * g
        eb_ref[...] = jnp.broadcast_to(ex, (B, 128))

    return pl.pallas_call(
        body,
        grid=(n,),
        in_specs=[
            pl.BlockSpec((B, 128), lambda i: (i, 0)),
            pl.BlockSpec((128, 1), lambda i: (0, 0)),
            pl.BlockSpec((1, 1), lambda i: (0, 0)),
        ],
        out_specs=[
            pl.BlockSpec((B, 128), lambda i: (i, 0)),
            pl.BlockSpec((B, 128), lambda i: (i, 0)),
        ],
        out_shape=[
            jax.ShapeDtypeStruct((G.shape[0], 128), jnp.float32),
            jax.ShapeDtypeStruct((G.shape[0], 128), jnp.float32),
        ],
    )(G, W1, b1)


def _tc_edge_norm(Sg, Sb, gamma, beta):
    M = Sg.shape[1]
    B = 400
    n = M // B

    def body(sg_ref, sb_ref, g_ref, be_ref, t2_ref):
        num = sg_ref[0] + sg_ref[1]
        s1 = sb_ref[0, :, 0:1] + sb_ref[1, :, 0:1]
        mu = jnp.mean(num, axis=1, keepdims=True)
        xc = num - mu
        var = jnp.mean(xc * xc, axis=1, keepdims=True)
        y = xc * lax.rsqrt(var + 1e-5) * g_ref[...] + be_ref[...]
        t2_ref[:, :128] = y
        t2_ref[:, 128:256] = jnp.broadcast_to(s1, (B, 128))

    return pl.pallas_call(
        body,
        grid=(n,),
        in_specs=[
            pl.BlockSpec((2, B, 128), lambda i: (0, i, 0)),
            pl.BlockSpec((2, B, 128), lambda i: (0, i, 0)),
            pl.BlockSpec((1, 128), lambda i: (0, 0)),
            pl.BlockSpec((1, 128), lambda i: (0, 0)),
        ],
        out_specs=pl.BlockSpec((B, 256), lambda i: (i, 0)),
        out_shape=jax.ShapeDtypeStruct((M, 256), jnp.float32),
    )(Sg, Sb, gamma, beta)


def _tc_logits2(EG, XA, W2, b2):
    B = 2000
    n = EG.shape[0] // B

    def body(eg_ref, xa_ref, w_ref, b_ref, r2_ref, rb_ref):
        xev = xa_ref[:, :128]
        s1g = xa_ref[:, 128:129] + 1e-16
        u = eg_ref[...]
        w = w_ref[...]
        a2 = (
            jnp.dot(xev, w[:128, :], preferred_element_type=jnp.float32)
            + jnp.dot(u, w[128:, :], preferred_element_type=jnp.float32) / s1g
            + b_ref[0, 0]
        )
        ex2 = jnp.exp(a2)
        r2_ref[...] = ex2 * xev
        rb_ref[...] = jnp.broadcast_to(ex2, (B, 128))

    return pl.pallas_call(
        body,
        grid=(n,),
        in_specs=[
            pl.BlockSpec((B, 128), lambda i: (i, 0)),
            pl.BlockSpec((B, 256), lambda i: (i, 0)),
            pl.BlockSpec((256, 1), lambda i: (0, 0)),
            pl.BlockSpec((1, 1), lambda i: (0, 0)),
        ],
        out_specs=[
            pl.BlockSpec((B, 128), lambda i: (i, 0)),
            pl.BlockSpec((B, 128), lambda i: (i, 0)),
        ],
        out_shape=[
            jax.ShapeDtypeStruct((EG.shape[0], 128), jnp.float32),
            jax.ShapeDtypeStruct((EG.shape[0], 128), jnp.float32),
        ],
    )(EG, XA, W2, b2)


def _tc_final(Sv, SvB, Xv_in):
    N = Xv_in.shape[0]
    B = 400
    n = N // B

    def body(s_ref, sb_ref, xv_ref, out_ref):
        num = s_ref[0] + s_ref[1]
        den = sb_ref[0, :, 0:1] + sb_ref[1, :, 0:1] + 1e-16
        out_ref[...] = num / den + xv_ref[...]

    return pl.pallas_call(
        body,
        grid=(n,),
        in_specs=[
            pl.BlockSpec((2, B, 128), lambda i: (0, i, 0)),
            pl.BlockSpec((2, B, 128), lambda i: (0, i, 0)),
            pl.BlockSpec((B, 128), lambda i: (i, 0)),
        ],
        out_specs=pl.BlockSpec((B, 128), lambda i: (i, 0)),
        out_shape=jax.ShapeDtypeStruct((N, 128), jnp.float32),
    )(Sv, SvB, Xv_in)


def kernel(Xv_in, v, e, W1, b1, W2, b2, gamma, beta):
    N, D = Xv_in.shape
    NNZ = v.shape[0]
    M = 20000
    MH = M // 2
    bpw = NNZ // NW
    nch = bpw // CH

    e0 = jnp.where(e < MH, e, TRASH).reshape(NW, nch, CH)
    e1 = jnp.where(e >= MH, e - MH, TRASH).reshape(NW, nch, CH)
    v3 = v.reshape(NW, nch, CH)
    r0 = _chunk_ranges(e0)
    r1 = _chunk_ranges(e1)
    rv = _chunk_ranges(v3)

    G = _sc_gather(Xv_in, v)
    EG, EB = _tc_logits1(G, W1, jnp.reshape(b1, (1, 1)))

    Sg = jnp.concatenate(
        [_sc_scatter_add(EG, e0, r0)[:, :MH],
         _sc_scatter_add(EG, e1, r1)[:, :MH]], axis=1)
    Sb = jnp.concatenate(
        [_sc_scatter_add(EB, e0, r0)[:, :MH],
         _sc_scatter_add(EB, e1, r1)[:, :MH]], axis=1)

    T2 = _tc_edge_norm(Sg, Sb, jnp.reshape(gamma, (1, 128)),
                       jnp.reshape(beta, (1, 128)))
    XA = _sc_gather(T2, e)
    R2, R2B = _tc_logits2(EG, XA, jnp.reshape(W2, (256, 1)),
                          jnp.reshape(b2, (1, 1)))

    Sv = _sc_scatter_add(R2, v3, rv)[:, :N]
    SvB = _sc_scatter_add(R2B, v3, rv)[:, :N]
    return _tc_final(Sv, SvB, Xv_in)
